# Initial kernel scaffold; baseline (speedup 1.0000x reference)
import functools

import jax
import jax.numpy as jnp
from jax import lax
from jax.experimental import pallas as pl
from jax.experimental.pallas import tpu as pltpu
from jax.experimental.pallas import tpu_sc as plsc

_PAIRS_A = (0, 2, 4, 6, 8, 10, 12, 14, 16, 18, 20, 22, 24, 0, 1, 4)
_PAIRS_B = (1, 3, 5, 7, 9, 11, 13, 15, 17, 19, 21, 23, 25, 2, 3, 6)

_I = 1000
_D = 26
_B = 16384
_P = 16

_NC = 2
_NS = 16
_NW = _NC * _NS
_BW = _B // _NW
_GROUPS = _BW // 16
_QUARTERS = _BW // 128
_NROW = _P * _QUARTERS


def _sc_body(xT_hbm, singles_hbm, pairs_hbm, out_hbm,
             xT_v, singles_v, pidx_v, prow_v, out_v, sem):
    wid = lax.axis_index("s") * _NC + lax.axis_index("c")
    base = wid * _BW

    pltpu.sync_copy(xT_hbm.at[:, pl.ds(base, _BW)], xT_v)
    pltpu.sync_copy(singles_hbm, singles_v)

    for p in range(_P):
        ra, rb = _PAIRS_A[p], _PAIRS_B[p]
        for q in range(_QUARTERS):
            row = p * _QUARTERS + q

            def build(c, _, row=row, ra=ra, rb=rb, q=q, p=p):
                b0 = q * 128 + c * 16
                ia = xT_v[ra, pl.ds(b0, 16)]
                ib = xT_v[rb, pl.ds(b0, 16)]
                pidx_v[row, pl.ds(c * 16, 16)] = p * (_I * _I) + ia * _I + ib
                return 0

            lax.fori_loop(0, 8, build, 0)

    def fire(j, _):
        pltpu.make_async_copy(pairs_hbm.at[pidx_v.at[j]], prow_v.at[j], sem).start()
        return 0

    lax.fori_loop(0, _NROW, fire, 0)

    def singles_acc(g, _):
        b0 = g * 16
        acc = jnp.zeros((16,), jnp.float32)
        for d in range(_D):
            xv = xT_v[d, pl.ds(b0, 16)]
            acc = acc + plsc.load_gather(singles_v, [xv + d * _I])
        out_v[pl.ds(b0, 16)] = acc
        return 0

    lax.fori_loop(0, _GROUPS, singles_acc, 0)

    def drain(j, _):
        pltpu.make_async_copy(pairs_hbm.at[pidx_v.at[j]], prow_v.at[j], sem).wait()
        return 0

    lax.fori_loop(0, _NROW, drain, 0)

    for j in range(_NROW):
        q = j % _QUARTERS

        def pairs_acc(c, _, j=j, q=q):
            sl = pl.ds(q * 128 + c * 16, 16)
            out_v[sl] = out_v[sl] + prow_v[j, pl.ds(c * 16, 16)]
            return 0

        lax.fori_loop(0, 8, pairs_acc, 0)

    pltpu.sync_copy(out_v, out_hbm.at[pl.ds(base, _BW)])


_sc_call = functools.partial(
    pl.kernel,
    mesh=plsc.VectorSubcoreMesh(core_axis_name="c", subcore_axis_name="s"),
    out_type=jax.ShapeDtypeStruct((_B,), jnp.float32),
    scratch_types=[
        pltpu.VMEM((_D, _BW), jnp.int32),
        pltpu.VMEM((_D * _I,), jnp.float32),
        pltpu.VMEM((_NROW, 128), jnp.int32),
        pltpu.VMEM((_NROW, 128), jnp.float32),
        pltpu.VMEM((_BW,), jnp.float32),
        pltpu.SemaphoreType.DMA,
    ],
)(_sc_body)


@jax.jit
def kernel(x, theta0, theta_singles, theta_pairs):
    xT = x.T.astype(jnp.int32)
    singles = theta_singles.reshape(-1).astype(jnp.float32)
    pairs = theta_pairs.reshape(-1).astype(jnp.float32)
    out = _sc_call(xT, singles, pairs)
    return out + theta0.astype(jnp.float32)


# speedup vs baseline: 36.7396x; 36.7396x over previous
import functools

import jax
import jax.numpy as jnp
from jax import lax
from jax.experimental import pallas as pl
from jax.experimental.pallas import tpu as pltpu
from jax.experimental.pallas import tpu_sc as plsc

_PAIRS_A = (0, 2, 4, 6, 8, 10, 12, 14, 16, 18, 20, 22, 24, 0, 1, 4)
_PAIRS_B = (1, 3, 5, 7, 9, 11, 13, 15, 17, 19, 21, 23, 25, 2, 3, 6)

_I = 1000
_D = 26
_B = 16384
_P = 16

_NC = 2
_NS = 16
_NW = _NC * _NS
_BW = _B // _NW
_GROUPS = _BW // 16
_QUARTERS = _BW // 128
_NROW = _P * _QUARTERS


def _sc_body(xT_hbm, singles_hbm, pairs_hbm, out_hbm,
             xT_v, singles_v, pidx_v, prow_v, out_v, sem):
    wid = lax.axis_index("s") * _NC + lax.axis_index("c")
    base = wid * _BW

    pltpu.sync_copy(xT_hbm.at[:, pl.ds(base, _BW)], xT_v)
    pltpu.sync_copy(singles_hbm, singles_v)

    for p in range(_P):
        ra, rb = _PAIRS_A[p], _PAIRS_B[p]
        for q in range(_QUARTERS):
            row = p * _QUARTERS + q

            def build(c, _, row=row, ra=ra, rb=rb, q=q, p=p):
                b0 = q * 128 + c * 16
                ia = xT_v[ra, pl.ds(b0, 16)]
                ib = xT_v[rb, pl.ds(b0, 16)]
                pidx_v[row, pl.ds(c * 16, 16)] = p * (_I * _I) + ia * _I + ib
                return 0

            lax.fori_loop(0, 8, build, 0)

    def fire(j, _):
        pltpu.make_async_copy(pairs_hbm.at[pidx_v.at[j]], prow_v.at[j], sem).start()
        return 0

    lax.fori_loop(0, _NROW, fire, 0)

    def singles_acc(g, _):
        b0 = g * 16
        acc = jnp.zeros((16,), jnp.float32)
        for d in range(_D):
            xv = xT_v[d, pl.ds(b0, 16)]
            acc = acc + plsc.load_gather(singles_v, [xv + d * _I])
        out_v[pl.ds(b0, 16)] = acc
        return 0

    lax.fori_loop(0, _GROUPS, singles_acc, 0)

    def drain(j, _):
        pltpu.make_async_copy(pairs_hbm.at[pidx_v.at[j]], prow_v.at[j], sem).wait()
        return 0

    lax.fori_loop(0, _NROW, drain, 0)

    for j in range(_NROW):
        q = j % _QUARTERS

        def pairs_acc(c, _, j=j, q=q):
            sl = pl.ds(q * 128 + c * 16, 16)
            out_v[sl] = out_v[sl] + prow_v[j, pl.ds(c * 16, 16)]
            return 0

        lax.fori_loop(0, 8, pairs_acc, 0)

    pltpu.sync_copy(out_v, out_hbm.at[pl.ds(base, _BW)])


_sc_call = functools.partial(
    pl.kernel,
    mesh=plsc.VectorSubcoreMesh(core_axis_name="c", subcore_axis_name="s"),
    out_type=jax.ShapeDtypeStruct((_B,), jnp.float32),
    compiler_params=pltpu.CompilerParams(needs_layout_passes=False),
    scratch_types=[
        pltpu.VMEM((_D, _BW), jnp.int32),
        pltpu.VMEM((_D * _I,), jnp.float32),
        pltpu.VMEM((_NROW, 128), jnp.int32),
        pltpu.VMEM((_NROW, 128), jnp.float32),
        pltpu.VMEM((_BW,), jnp.float32),
        pltpu.SemaphoreType.DMA,
    ],
)(_sc_body)


@jax.jit
def kernel(x, theta0, theta_singles, theta_pairs):
    xT = x.T.astype(jnp.int32)
    singles = theta_singles.reshape(-1).astype(jnp.float32)
    pairs = theta_pairs.reshape(-1).astype(jnp.float32)
    out = _sc_call(xT, singles, pairs)
    return out + theta0.astype(jnp.float32)
